# baseline (speedup 1.0000x reference)
import functools

import jax
import jax.numpy as jnp
from jax import lax
from jax.experimental import pallas as pl
from jax.experimental.pallas import tpu as pltpu
from jax.experimental.pallas import tpu_sc as plsc

_B, _S, _H = 4, 4096, 1024
_N = _B * _S
_NC, _NS = 2, 16
_NW = _NC * _NS
_PER_W = _N // _NW
_C = 32
_STEPS = _PER_W // _C
_L = 16

_mesh = plsc.VectorSubcoreMesh(core_axis_name="c", subcore_axis_name="s")


@functools.partial(
    pl.kernel,
    mesh=_mesh,
    out_type=jax.ShapeDtypeStruct((_N, _H), jnp.float32),
    scratch_types=[
        pltpu.VMEM((_STEPS, _C), jnp.int32),
        pltpu.VMEM((_C, _H), jnp.float32),
        pltpu.VMEM((_C, _H), jnp.float32),
        pltpu.SemaphoreType.DMA,
    ],
)
def _embed_add(emb_hbm, idx_hbm, table_hbm, out_hbm, idx_v, emb_v, rows_v, sem):
    wid = lax.axis_index("s") * _NC + lax.axis_index("c")
    base = wid * _PER_W
    pltpu.sync_copy(idx_hbm.at[wid], idx_v)

    def step(j, carry):
        off = base + j * _C
        gather = pltpu.async_copy(table_hbm.at[idx_v.at[j]], rows_v, sem)
        pltpu.sync_copy(emb_hbm.at[pl.ds(off, _C)], emb_v)
        gather.wait()

        def row(r, c2):
            def col(k, c3):
                sl = pl.ds(k * _L, _L)
                emb_v[r, sl] = emb_v[r, sl] + rows_v[r, sl]
                return c3
            return lax.fori_loop(0, _H // _L, col, c2)

        lax.fori_loop(0, _C, row, 0)
        pltpu.sync_copy(emb_v, out_hbm.at[pl.ds(off, _C)])
        return carry

    lax.fori_loop(0, _STEPS, step, 0)


def kernel(embeddings, position_ids, table):
    emb = embeddings.reshape(_N, _H)
    idx = position_ids.reshape(_NW, _STEPS, _C).astype(jnp.int32)
    out = _embed_add(emb, idx, table)
    return out.reshape(_B, _S, _H)


# speedup vs baseline: 2.6218x; 2.6218x over previous
import functools

import jax
import jax.numpy as jnp
from jax import lax
from jax.experimental import pallas as pl
from jax.experimental.pallas import tpu as pltpu
from jax.experimental.pallas import tpu_sc as plsc

_B, _S, _H = 4, 4096, 1024
_N = _B * _S
_NC, _NS = 2, 16
_NW = _NC * _NS
_PER_W = _N // _NW
_C = 16
_STEPS = _PER_W // _C
_L = 16

_mesh = plsc.VectorSubcoreMesh(core_axis_name="c", subcore_axis_name="s")


@functools.partial(
    pl.kernel,
    mesh=_mesh,
    out_type=jax.ShapeDtypeStruct((_N, _H), jnp.float32),
    scratch_types=[
        pltpu.VMEM((_STEPS, _C), jnp.int32),
        pltpu.VMEM((_C, _H), jnp.float32),
        pltpu.VMEM((_C, _H), jnp.float32),
        pltpu.VMEM((_C, _H), jnp.float32),
        pltpu.VMEM((_C, _H), jnp.float32),
        pltpu.VMEM((_C, _H), jnp.float32),
        pltpu.VMEM((_C, _H), jnp.float32),
        pltpu.SemaphoreType.DMA,
        pltpu.SemaphoreType.DMA,
        pltpu.SemaphoreType.DMA,
        pltpu.SemaphoreType.DMA,
        pltpu.SemaphoreType.DMA,
        pltpu.SemaphoreType.DMA,
    ],
)
def _embed_add(emb_hbm, idx_hbm, table_hbm, out_hbm,
               idx_v, e0, e1, r0, r1, s0, s1,
               se0, se1, sr0, sr1, sw0, sw1):
    wid = lax.axis_index("s") * _NC + lax.axis_index("c")
    base = wid * _PER_W
    pltpu.sync_copy(idx_hbm.at[wid], idx_v)

    def issue_loads(j, eb, rb, sem_e, sem_r):
        off = base + j * _C
        pltpu.async_copy(table_hbm.at[idx_v.at[j]], rb, sem_r)
        pltpu.async_copy(emb_hbm.at[pl.ds(off, _C)], eb, sem_e)

    def process(j, eb, rb, sb, sem_e, sem_r, sem_w):
        off = base + j * _C
        pltpu.make_async_copy(emb_hbm.at[pl.ds(off, _C)], rb, sem_r).wait()
        pltpu.make_async_copy(emb_hbm.at[pl.ds(off, _C)], eb, sem_e).wait()

        @pl.when(j >= 2)
        def _():
            pltpu.make_async_copy(sb, out_hbm.at[pl.ds(off, _C)], sem_w).wait()

        def row(r, c):
            for k in range(_H // _L):
                sl = pl.ds(k * _L, _L)
                sb[r, sl] = eb[r, sl] + rb[r, sl]
            return c

        lax.fori_loop(0, _C, row, 0)
        pltpu.async_copy(sb, out_hbm.at[pl.ds(off, _C)], sem_w)

        @pl.when(j + 2 < _STEPS)
        def _():
            issue_loads(j + 2, eb, rb, sem_e, sem_r)

    issue_loads(0, e0, r0, se0, sr0)
    issue_loads(1, e1, r1, se1, sr1)

    def body(i, c):
        process(2 * i, e0, r0, s0, se0, sr0, sw0)
        process(2 * i + 1, e1, r1, s1, se1, sr1, sw1)
        return c

    lax.fori_loop(0, _STEPS // 2, body, 0)

    pltpu.make_async_copy(s0, out_hbm.at[pl.ds(base, _C)], sw0).wait()
    pltpu.make_async_copy(s1, out_hbm.at[pl.ds(base, _C)], sw1).wait()


def kernel(embeddings, position_ids, table):
    emb = embeddings.reshape(_N, _H)
    idx = position_ids.reshape(_NW, _STEPS, _C).astype(jnp.int32)
    out = _embed_add(emb, idx, table)
    return out.reshape(_B, _S, _H)
